# baseline (speedup 1.0000x reference)
import functools

import jax
import jax.numpy as jnp
from jax import lax
from jax.experimental import pallas as pl
from jax.experimental.pallas import tpu as pltpu
from jax.experimental.pallas import tpu_sc as plsc

_NUM_USERS = 1000000
_NUM_ITEMS = 1000
_DIM = 32
_BATCH = 16384

_NC = 2
_NS = 16
_NW = _NC * _NS
_BPW = _BATCH // _NW
_CHUNK = 128
_NCHUNK = _BPW // _CHUNK

_BM = 512


@functools.cache
def _build_sc_gather():
    mesh = plsc.VectorSubcoreMesh(
        core_axis_name="c", subcore_axis_name="s",
        num_cores=_NC, num_subcores=_NS,
    )

    @functools.partial(
        pl.kernel,
        mesh=mesh,
        out_type=jax.ShapeDtypeStruct((_BATCH, _DIM), jnp.float32),
        scratch_types=[
            pltpu.VMEM((_BPW,), jnp.int32),
            pltpu.VMEM((_BPW, _DIM), jnp.float32),
            pltpu.SemaphoreType.DMA,
        ],
        compiler_params=pltpu.CompilerParams(use_tc_tiling_on_sc=False),
    )
    def sc_gather(table_hbm, idx_hbm, out_hbm, idx_v, rows_v, sem):
        wid = lax.axis_index("s") * _NC + lax.axis_index("c")
        base = wid * _BPW
        pltpu.sync_copy(idx_hbm.at[pl.ds(base, _BPW)], idx_v)
        copies = []
        for j in range(_NCHUNK):
            copies.append(
                pltpu.async_copy(
                    table_hbm.at[idx_v.at[pl.ds(j * _CHUNK, _CHUNK)]],
                    rows_v.at[pl.ds(j * _CHUNK, _CHUNK)],
                    sem,
                )
            )
        for c in copies:
            c.wait()
        pltpu.sync_copy(rows_v, out_hbm.at[pl.ds(base, _BPW)])

    return sc_gather


def _tc_body(theta_ref, alpha_ref, out_ref):
    theta = theta_ref[...]
    alpha = alpha_ref[...]
    u = lax.dot_general(
        theta, alpha, (((1,), (1,)), ((), ())),
        preferred_element_type=jnp.float32,
    )
    m = jnp.max(u, axis=-1, keepdims=True)
    e = jnp.exp(u - m)
    s = jnp.sum(e, axis=-1, keepdims=True)
    out_ref[...] = (u - m) - jnp.log(s)


@functools.cache
def _build_tc_logits():
    return pl.pallas_call(
        _tc_body,
        grid=(_BATCH // _BM,),
        in_specs=[
            pl.BlockSpec((_BM, _DIM), lambda i: (i, 0)),
            pl.BlockSpec((_NUM_ITEMS, _DIM), lambda i: (0, 0)),
        ],
        out_specs=pl.BlockSpec((_BM, _NUM_ITEMS), lambda i: (i, 0)),
        out_shape=jax.ShapeDtypeStruct((_BATCH, _NUM_ITEMS), jnp.float32),
    )


def kernel(user_index, theta_user, alpha_item):
    theta = _build_sc_gather()(theta_user, user_index)
    return _build_tc_logits()(theta, alpha_item)
